# speedup vs baseline: 2.7067x; 2.7067x over previous
import jax
import jax.numpy as jnp
from jax.experimental import pallas as pl
from jax.experimental.pallas import tpu as pltpu

_T = 4
_TB = 512


def _body(x0, x1, x2, x3, w, bias, al, out):
    f32 = jnp.float32
    logits = (
        jax.lax.dot_general(x0[...], w[0], (((1,), (0,)), ((), ())),
                            preferred_element_type=f32)
        + jax.lax.dot_general(x1[...], w[1], (((1,), (0,)), ((), ())),
                              preferred_element_type=f32)
        + jax.lax.dot_general(x2[...], w[2], (((1,), (0,)), ((), ())),
                              preferred_element_type=f32)
        + jax.lax.dot_general(x3[...], w[3], (((1,), (0,)), ((), ())),
                              preferred_element_type=f32)
    ) + bias[0, :]
    lt = logits[:, :_T]
    ls = logits[:, _T:]

    col = jax.lax.broadcasted_iota(jnp.int32, (_TB, _T), 1)
    v1 = jnp.max(lt, axis=-1, keepdims=True)
    i1 = jnp.min(jnp.where(lt >= v1, col, _T), axis=-1, keepdims=True)
    m1 = col == i1
    lt2 = jnp.where(m1, -jnp.inf, lt)
    v2 = jnp.max(lt2, axis=-1, keepdims=True)
    i2 = jnp.min(jnp.where(lt2 >= v2, col, _T), axis=-1, keepdims=True)
    m2 = col == i2

    e2 = jnp.exp(v2 - v1)
    p1 = 1.0 / (1.0 + e2)
    type_w = jnp.where(m1, p1, 0.0) + jnp.where(m2, 1.0 - p1, 0.0)

    es = jnp.exp(ls - jnp.max(ls, axis=-1, keepdims=True))
    soft = es / jnp.sum(es, axis=-1, keepdims=True)

    a = jax.nn.sigmoid(al[0, 0])
    wts = a * type_w + (1.0 - a) * soft

    out[...] = (x0[...] * wts[:, 0:1] + x1[...] * wts[:, 1:2]
                + x2[...] * wts[:, 2:3] + x3[...] * wts[:, 3:4])


def kernel(mod0, mod1, mod2, mod3, W_top, b_top, W_soft, b_soft, alpha):
    B, S, D = mod0.shape
    N = B * S
    xs = [m.reshape(N, D) for m in (mod0, mod1, mod2, mod3)]

    wt = W_top.reshape(_T, D, _T).transpose(2, 1, 0)
    ws = W_soft.reshape(_T, D, _T).transpose(2, 1, 0)
    w = jnp.concatenate([wt, ws], axis=-1)
    bias = jnp.concatenate([b_top, b_soft]).reshape(1, 2 * _T)
    al = alpha.reshape(1, 1)

    grid = (N // _TB,)
    xspec = pl.BlockSpec((_TB, D), lambda i: (i, 0))
    full = lambda *s: pl.BlockSpec(s, lambda i: tuple(0 for _ in s))
    out = pl.pallas_call(
        _body,
        grid=grid,
        in_specs=[xspec, xspec, xspec, xspec,
                  full(_T, D, 2 * _T), full(1, 2 * _T), full(1, 1)],
        out_specs=xspec,
        out_shape=jax.ShapeDtypeStruct((N, D), jnp.float32),
        compiler_params=pltpu.CompilerParams(
            dimension_semantics=("arbitrary",)),
    )(xs[0], xs[1], xs[2], xs[3], w, bias, al)
    return out.reshape(B, S, D)
